# Initial kernel scaffold; baseline (speedup 1.0000x reference)
import jax
import jax.numpy as jnp
from jax.experimental import pallas as pl


def kernel(code, centroid):
    raise NotImplementedError("write your pallas kernel here")


# speedup vs baseline: 3.7151x; 3.7151x over previous
import functools

import jax
import jax.numpy as jnp
from jax import lax
from jax.experimental import pallas as pl
from jax.experimental.pallas import tpu as pltpu
from jax.experimental.pallas import tpu_sc as plsc

NUM_SUB = 8
K = 8192
SUB_DIM = 64
NUM_CODES = 100000
B = NUM_CODES * NUM_SUB
CHUNK = 800
NUM_CHUNKS = B // CHUNK
LANES = 16


def _make_gather_kernel():
    info = plsc.get_sparse_core_info()
    nc, ns = info.num_cores, info.num_subcores
    nw = nc * ns
    mesh = plsc.VectorSubcoreMesh(core_axis_name="c", subcore_axis_name="s")

    @functools.partial(
        pl.kernel,
        out_type=jax.ShapeDtypeStruct((B, SUB_DIM), jnp.float32),
        mesh=mesh,
        scratch_types=[
            pltpu.VMEM((CHUNK,), jnp.int32),
            pltpu.VMEM((CHUNK, SUB_DIM), jnp.float32),
            pltpu.SemaphoreType.DMA,
        ],
        compiler_params=pltpu.CompilerParams(use_tc_tiling_on_sc=False),
    )
    def gather_kernel(table_hbm, idx_hbm, out_hbm, idx_v, rows_v, sem):
        wid = lax.axis_index("s") * nc + lax.axis_index("c")
        offs = (lax.broadcasted_iota(jnp.int32, (LANES,), 0) & 7) * K
        n_mine = (NUM_CHUNKS - wid + nw - 1) // nw

        def chunk_body(t, carry):
            base = (wid + t * nw) * CHUNK
            pltpu.sync_copy(idx_hbm.at[pl.ds(base, CHUNK)], idx_v)

            def add_offs(g, c):
                sl = pl.ds(g * LANES, LANES)
                idx_v[sl] = idx_v[sl] + offs
                return c

            lax.fori_loop(0, CHUNK // LANES, add_offs, 0, unroll=True)
            pltpu.async_copy(table_hbm.at[idx_v], rows_v, sem).wait()
            pltpu.sync_copy(rows_v, out_hbm.at[pl.ds(base, CHUNK)])
            return carry

        lax.fori_loop(0, n_mine, chunk_body, 0)

    return gather_kernel


_gather = _make_gather_kernel()


@jax.jit
def kernel(code, centroid):
    table = centroid.reshape(NUM_SUB * K, SUB_DIM)
    idx = code.astype(jnp.int32).reshape(B)
    rows = _gather(table, idx)
    return rows.reshape(NUM_CODES, NUM_SUB * SUB_DIM)


# speedup vs baseline: 3.9958x; 1.0755x over previous
import functools

import jax
import jax.numpy as jnp
from jax import lax
from jax.experimental import pallas as pl
from jax.experimental.pallas import tpu as pltpu
from jax.experimental.pallas import tpu_sc as plsc

NUM_SUB = 8
K = 8192
SUB_DIM = 64
NUM_CODES = 100000
B = NUM_CODES * NUM_SUB
CHUNK = 800
NUM_CHUNKS = B // CHUNK
LANES = 16


def _make_gather_kernel():
    info = plsc.get_sparse_core_info()
    nc, ns = info.num_cores, info.num_subcores
    nw = nc * ns
    max_mine = -(-NUM_CHUNKS // nw)
    n_pairs = -(-max_mine // 2)
    mesh = plsc.VectorSubcoreMesh(core_axis_name="c", subcore_axis_name="s")

    @functools.partial(
        pl.kernel,
        out_type=jax.ShapeDtypeStruct((B, SUB_DIM), jnp.float32),
        mesh=mesh,
        scratch_types=[
            pltpu.VMEM((CHUNK,), jnp.int32),
            pltpu.VMEM((CHUNK,), jnp.int32),
            pltpu.VMEM((CHUNK, SUB_DIM), jnp.float32),
            pltpu.VMEM((CHUNK, SUB_DIM), jnp.float32),
            pltpu.SemaphoreType.DMA,
            pltpu.SemaphoreType.DMA,
            pltpu.SemaphoreType.DMA,
            pltpu.SemaphoreType.DMA,
        ],
        compiler_params=pltpu.CompilerParams(use_tc_tiling_on_sc=False),
    )
    def gather_kernel(table_hbm, idx_hbm, out_hbm,
                      idx0, idx1, rows0, rows1, g0, g1, w0, w1):
        wid = lax.axis_index("s") * nc + lax.axis_index("c")
        idx_b, rows_b = (idx0, idx1), (rows0, rows1)
        gsem, wsem = (g0, g1), (w0, w1)
        offs = (lax.broadcasted_iota(jnp.int32, (LANES,), 0) & 7) * K
        n_mine = (NUM_CHUNKS - wid + nw - 1) // nw

        def chunk_base(t):
            return (wid + t * nw) * CHUNK

        def load(t, b):
            pltpu.sync_copy(idx_hbm.at[pl.ds(chunk_base(t), CHUNK)], idx_b[b])

            def add_offs(g, c):
                sl = pl.ds(g * LANES, LANES)
                idx_b[b][sl] = idx_b[b][sl] + offs
                return c

            lax.fori_loop(0, CHUNK // LANES, add_offs, 0, unroll=True)

            @pl.when(t >= 2)
            def _():
                pltpu.make_async_copy(
                    rows_b[b], out_hbm.at[pl.ds(0, CHUNK)], wsem[b]).wait()

            pltpu.async_copy(table_hbm.at[idx_b[b]], rows_b[b], gsem[b])

        def store(t, b):
            pltpu.make_async_copy(
                table_hbm.at[idx_b[b]], rows_b[b], gsem[b]).wait()
            pltpu.async_copy(
                rows_b[b], out_hbm.at[pl.ds(chunk_base(t), CHUNK)], wsem[b])

        load(0, 0)

        def pair(g, carry):
            t1 = 2 * g + 1

            @pl.when(t1 < n_mine)
            def _():
                load(t1, 1)

            store(2 * g, 0)

            @pl.when(t1 + 1 < n_mine)
            def _():
                load(t1 + 1, 0)

            @pl.when(t1 < n_mine)
            def _():
                store(t1, 1)

            return carry

        lax.fori_loop(0, n_pairs, pair, 0)
        for b in (0, 1):
            pltpu.make_async_copy(
                rows_b[b], out_hbm.at[pl.ds(0, CHUNK)], wsem[b]).wait()

    return gather_kernel


_gather = _make_gather_kernel()


@jax.jit
def kernel(code, centroid):
    table = centroid.reshape(NUM_SUB * K, SUB_DIM)
    idx = code.astype(jnp.int32).reshape(B)
    rows = _gather(table, idx)
    return rows.reshape(NUM_CODES, NUM_SUB * SUB_DIM)
